# Initial kernel scaffold; baseline (speedup 1.0000x reference)
import jax
import jax.numpy as jnp
from jax.experimental import pallas as pl


def kernel(xyz1, xyz2, points1, points2, feature, W0, b0, g0, beta0, W1, b1, g1, beta1):
    raise NotImplementedError("write your pallas kernel here")


# speedup vs baseline: 12.8385x; 12.8385x over previous
import jax
import jax.numpy as jnp
from jax.experimental import pallas as pl


def _k1_body(x1t_ref, x2_ref, ft_ref, feat_ref, p2t_ref, p1t_ref, w0t_ref,
             b0_ref, y1_ref, s_ref):
    TILE = x1t_ref.shape[1]
    S = x2_ref.shape[2]
    f32 = jnp.float32
    HI = jax.lax.Precision.HIGHEST
    x1 = x1t_ref[0]
    x2 = x2_ref[0]
    x1b = x1.astype(jnp.bfloat16)
    x1sq = jnp.sum(x1 * x1, axis=1, keepdims=True)
    x2sq = jnp.sum(x2 * x2, axis=0, keepdims=True)
    cross = jnp.dot(x1b, x2.astype(jnp.bfloat16),
                    preferred_element_type=f32)
    d = -2.0 * cross
    d = d + x1sq
    d = d + x2sq

    iota = jax.lax.broadcasted_iota(jnp.int32, (TILE, S), 1)
    big = jnp.float32(jnp.inf)
    vals = []
    idxs = []
    dcur = d
    for _ in range(3):
        m = jnp.min(dcur, axis=1, keepdims=True)
        im = jnp.min(jnp.where(dcur == m, iota, S), axis=1,
                     keepdims=True)
        vals.append(m)
        idxs.append(im)
        dcur = jnp.where(iota == im, big, dcur)

    r0 = 1.0 / (vals[0] + 1e-8)
    r1 = 1.0 / (vals[1] + 1e-8)
    r2 = 1.0 / (vals[2] + 1e-8)
    norm = r0 + r1 + r2
    w0 = r0 / norm
    w1 = r1 / norm
    w2 = r2 / norm
    zero = jnp.float32(0.0)
    wmat = (jnp.where(iota == idxs[0], w0, zero)
            + jnp.where(iota == idxs[1], w1, zero)
            + jnp.where(iota == idxs[2], w2, zero))
    interp = jnp.dot(wmat, p2t_ref[0], precision=HI,
                     preferred_element_type=f32)

    ft = ft_ref[0]
    fc = ft[1:4, :]
    fcsq = jnp.sum(fc * fc, axis=0, keepdims=True)
    cross2 = jnp.dot(x1b, fc.astype(jnp.bfloat16),
                     preferred_element_type=f32)
    d2 = -2.0 * cross2
    d2 = d2 + x1sq
    d2 = d2 + fcsq
    m1 = jnp.min(d2, axis=1, keepdims=True)
    i1 = jnp.min(jnp.where(d2 == m1, iota, S), axis=1, keepdims=True)
    oh = jnp.where(iota == i1, jnp.float32(1.0), zero)
    nf = jnp.dot(oh, feat_ref[0], precision=HI,
                 preferred_element_type=f32)

    np_t = jnp.concatenate([p1t_ref[0], interp, nf], axis=1)
    y1 = jnp.dot(np_t, w0t_ref[...],
                 preferred_element_type=f32) + b0_ref[...]
    y1_ref[0] = y1

    s = jnp.sum(y1, axis=0, keepdims=True)
    ss = jnp.sum(y1 * y1, axis=0, keepdims=True)
    pad = jnp.zeros((6, y1.shape[1]), jnp.float32)
    s_ref[0, 0] = jnp.concatenate([s, ss, pad], axis=0)


def _k2_body(y1_ref, a1_ref, c1_ref, w1t_ref, b1_ref, y2_ref, s_ref):
    z = jnp.maximum(y1_ref[0] * a1_ref[...] + c1_ref[...], 0.0)
    y2 = jnp.dot(z, w1t_ref[...],
                 preferred_element_type=jnp.float32) + b1_ref[...]
    y2_ref[0] = y2
    s = jnp.sum(y2, axis=0, keepdims=True)
    ss = jnp.sum(y2 * y2, axis=0, keepdims=True)
    pad = jnp.zeros((6, y2.shape[1]), jnp.float32)
    s_ref[0, 0] = jnp.concatenate([s, ss, pad], axis=0)


def _ks_body(cnt, p_ref, g_ref, be_ref, a_ref, c_ref):
    r = p_ref[...]
    R8, O = r.shape
    row = jax.lax.broadcasted_iota(jnp.int32, (R8, O), 0)
    zero = jnp.float32(0.0)
    s = jnp.sum(jnp.where(row % 8 == 0, r, zero), axis=0, keepdims=True)
    ss = jnp.sum(jnp.where(row % 8 == 1, r, zero), axis=0, keepdims=True)
    m = s / cnt
    v = ss / cnt - m * m
    a = g_ref[...] / jnp.sqrt(v + 1e-5)
    a_ref[...] = a
    c_ref[...] = be_ref[...] - m * a


def _k3_body(y2_ref, a2_ref, c2_ref, out_ref):
    z = jnp.maximum(y2_ref[0] * a2_ref[...] + c2_ref[...], 0.0)
    out_ref[0] = jnp.transpose(z, (1, 0))


def kernel(xyz1, xyz2, points1, points2, feature, W0, b0, g0, beta0,
           W1, b1, g1, beta1):
    B, _, N = xyz1.shape
    S = xyz2.shape[2]
    D1 = points1.shape[1]
    D2 = points2.shape[1]
    O0 = W0.shape[0]
    O1 = W1.shape[0]
    CIN = D1 + D2 + 4
    TILE = min(256, N)
    f32 = jnp.float32

    x1t = jnp.transpose(xyz1, (0, 2, 1))
    p1t = jnp.transpose(points1, (0, 2, 1))
    p2t = jnp.transpose(points2, (0, 2, 1))
    ft = jnp.transpose(feature, (0, 2, 1))
    w0t = jnp.transpose(W0)
    w1t = jnp.transpose(W1)
    b0r = b0.reshape(1, O0)
    b1r = b1.reshape(1, O1)

    NJ = N // TILE
    grid = (B, NJ)

    y1, p1s = pl.pallas_call(
        _k1_body,
        grid=grid,
        in_specs=[
            pl.BlockSpec((1, TILE, 3), lambda i, j: (i, j, 0)),
            pl.BlockSpec((1, 3, S), lambda i, j: (i, 0, 0)),
            pl.BlockSpec((1, 4, S), lambda i, j: (i, 0, 0)),
            pl.BlockSpec((1, S, 4), lambda i, j: (i, 0, 0)),
            pl.BlockSpec((1, S, D2), lambda i, j: (i, 0, 0)),
            pl.BlockSpec((1, TILE, D1), lambda i, j: (i, j, 0)),
            pl.BlockSpec((CIN, O0), lambda i, j: (0, 0)),
            pl.BlockSpec((1, O0), lambda i, j: (0, 0)),
        ],
        out_specs=[
            pl.BlockSpec((1, TILE, O0), lambda i, j: (i, j, 0)),
            pl.BlockSpec((1, 1, 8, O0), lambda i, j: (i, j, 0, 0)),
        ],
        out_shape=[
            jax.ShapeDtypeStruct((B, N, O0), f32),
            jax.ShapeDtypeStruct((B, NJ, 8, O0), f32),
        ],
    )(x1t, xyz2, ft, feature, p2t, p1t, w0t, b0r)
    y1, p1s = jax.lax.optimization_barrier((y1, p1s))

    import functools
    cnt = float(B * N)
    R8 = B * NJ * 8

    def stats(parts, g, be, O):
        return pl.pallas_call(
            functools.partial(_ks_body, cnt),
            in_specs=[
                pl.BlockSpec((R8, O), lambda: (0, 0)),
                pl.BlockSpec((1, O), lambda: (0, 0)),
                pl.BlockSpec((1, O), lambda: (0, 0)),
            ],
            out_specs=[
                pl.BlockSpec((1, O), lambda: (0, 0)),
                pl.BlockSpec((1, O), lambda: (0, 0)),
            ],
            out_shape=[
                jax.ShapeDtypeStruct((1, O), f32),
                jax.ShapeDtypeStruct((1, O), f32),
            ],
        )(parts.reshape(R8, O), g.reshape(1, O), be.reshape(1, O))

    a1, c1 = jax.lax.optimization_barrier(stats(p1s, g0, beta0, O0))

    y2, p2s = pl.pallas_call(
        _k2_body,
        grid=grid,
        in_specs=[
            pl.BlockSpec((1, TILE, O0), lambda i, j: (i, j, 0)),
            pl.BlockSpec((1, O0), lambda i, j: (0, 0)),
            pl.BlockSpec((1, O0), lambda i, j: (0, 0)),
            pl.BlockSpec((O0, O1), lambda i, j: (0, 0)),
            pl.BlockSpec((1, O1), lambda i, j: (0, 0)),
        ],
        out_specs=[
            pl.BlockSpec((1, TILE, O1), lambda i, j: (i, j, 0)),
            pl.BlockSpec((1, 1, 8, O1), lambda i, j: (i, j, 0, 0)),
        ],
        out_shape=[
            jax.ShapeDtypeStruct((B, N, O1), f32),
            jax.ShapeDtypeStruct((B, NJ, 8, O1), f32),
        ],
    )(y1, a1, c1, w1t, b1r)
    y2, p2s = jax.lax.optimization_barrier((y2, p2s))

    a2, c2 = jax.lax.optimization_barrier(stats(p2s, g1, beta1, O1))

    out = pl.pallas_call(
        _k3_body,
        grid=grid,
        in_specs=[
            pl.BlockSpec((1, TILE, O1), lambda i, j: (i, j, 0)),
            pl.BlockSpec((1, O1), lambda i, j: (0, 0)),
            pl.BlockSpec((1, O1), lambda i, j: (0, 0)),
        ],
        out_specs=pl.BlockSpec((1, O1, TILE), lambda i, j: (i, 0, j)),
        out_shape=jax.ShapeDtypeStruct((B, O1, N), f32),
    )(y2, a2, c2)

    return out


# speedup vs baseline: 15.6194x; 1.2166x over previous
import jax
import jax.numpy as jnp
from jax.experimental import pallas as pl


def _k1_body(x1t_ref, x2_ref, ft_ref, feat_ref, p2t_ref, p1t_ref, w0t_ref,
             b0_ref, y1_ref, s_ref):
    TILE = x1t_ref.shape[1]
    S = x2_ref.shape[2]
    f32 = jnp.float32
    HI = jax.lax.Precision.HIGHEST
    x1 = x1t_ref[0]
    x2 = x2_ref[0]
    x1b = x1.astype(jnp.bfloat16)
    x1sq = jnp.sum(x1 * x1, axis=1, keepdims=True)
    x2sq = jnp.sum(x2 * x2, axis=0, keepdims=True)
    cross = jnp.dot(x1b, x2.astype(jnp.bfloat16),
                    preferred_element_type=f32)
    d = -2.0 * cross
    d = d + x1sq
    d = d + x2sq

    iota = jax.lax.broadcasted_iota(jnp.int32, (TILE, S), 1)
    big = jnp.float32(jnp.inf)
    vals = []
    idxs = []
    dcur = d
    for _ in range(3):
        m = jnp.min(dcur, axis=1, keepdims=True)
        im = jnp.min(jnp.where(dcur == m, iota, S), axis=1,
                     keepdims=True)
        vals.append(m)
        idxs.append(im)
        dcur = jnp.where(iota == im, big, dcur)

    r0 = 1.0 / (vals[0] + 1e-8)
    r1 = 1.0 / (vals[1] + 1e-8)
    r2 = 1.0 / (vals[2] + 1e-8)
    norm = r0 + r1 + r2
    zero = jnp.float32(0.0)
    wmat = jnp.where(dcur == big, (1.0 / (d + 1e-8)) / norm, zero)
    interp = jnp.dot(wmat, p2t_ref[0], precision=HI,
                     preferred_element_type=f32)

    ft = ft_ref[0]
    fc = ft[1:4, :]
    fcsq = jnp.sum(fc * fc, axis=0, keepdims=True)
    cross2 = jnp.dot(x1b, fc.astype(jnp.bfloat16),
                     preferred_element_type=f32)
    d2 = -2.0 * cross2
    d2 = d2 + x1sq
    d2 = d2 + fcsq
    m1 = jnp.min(d2, axis=1, keepdims=True)
    i1 = jnp.min(jnp.where(d2 == m1, iota, S), axis=1, keepdims=True)
    oh = jnp.where(iota == i1, jnp.float32(1.0), zero)
    nf = jnp.dot(oh, feat_ref[0], precision=HI,
                 preferred_element_type=f32)

    np_t = jnp.concatenate([p1t_ref[0], interp, nf], axis=1)
    y1 = jnp.dot(np_t, w0t_ref[...],
                 preferred_element_type=f32) + b0_ref[...]
    y1_ref[0] = y1

    s = jnp.sum(y1, axis=0, keepdims=True)
    ss = jnp.sum(y1 * y1, axis=0, keepdims=True)
    pad = jnp.zeros((6, y1.shape[1]), jnp.float32)
    s_ref[0, 0] = jnp.concatenate([s, ss, pad], axis=0)


def _k2_body(y1_ref, a1_ref, c1_ref, w1t_ref, b1_ref, y2_ref, s_ref):
    z = jnp.maximum(y1_ref[0] * a1_ref[...] + c1_ref[...], 0.0)
    y2 = jnp.dot(z, w1t_ref[...],
                 preferred_element_type=jnp.float32) + b1_ref[...]
    y2_ref[0] = y2
    s = jnp.sum(y2, axis=0, keepdims=True)
    ss = jnp.sum(y2 * y2, axis=0, keepdims=True)
    pad = jnp.zeros((6, y2.shape[1]), jnp.float32)
    s_ref[0, 0] = jnp.concatenate([s, ss, pad], axis=0)


def _ks_body(cnt, p_ref, g_ref, be_ref, a_ref, c_ref):
    r = p_ref[...]
    R8, O = r.shape
    row = jax.lax.broadcasted_iota(jnp.int32, (R8, O), 0)
    zero = jnp.float32(0.0)
    s = jnp.sum(jnp.where(row % 8 == 0, r, zero), axis=0, keepdims=True)
    ss = jnp.sum(jnp.where(row % 8 == 1, r, zero), axis=0, keepdims=True)
    m = s / cnt
    v = ss / cnt - m * m
    a = g_ref[...] / jnp.sqrt(v + 1e-5)
    a_ref[...] = a
    c_ref[...] = be_ref[...] - m * a


def _k3_body(y2_ref, a2_ref, c2_ref, out_ref):
    z = jnp.maximum(y2_ref[0] * a2_ref[...] + c2_ref[...], 0.0)
    out_ref[0] = jnp.transpose(z, (1, 0))


def kernel(xyz1, xyz2, points1, points2, feature, W0, b0, g0, beta0,
           W1, b1, g1, beta1):
    B, _, N = xyz1.shape
    S = xyz2.shape[2]
    D1 = points1.shape[1]
    D2 = points2.shape[1]
    O0 = W0.shape[0]
    O1 = W1.shape[0]
    CIN = D1 + D2 + 4
    TILE = min(512, N)
    f32 = jnp.float32

    x1t = jnp.transpose(xyz1, (0, 2, 1))
    p1t = jnp.transpose(points1, (0, 2, 1))
    p2t = jnp.transpose(points2, (0, 2, 1))
    ft = jnp.transpose(feature, (0, 2, 1))
    w0t = jnp.transpose(W0)
    w1t = jnp.transpose(W1)
    b0r = b0.reshape(1, O0)
    b1r = b1.reshape(1, O1)

    NJ = N // TILE
    grid = (B, NJ)

    y1, p1s = pl.pallas_call(
        _k1_body,
        grid=grid,
        in_specs=[
            pl.BlockSpec((1, TILE, 3), lambda i, j: (i, j, 0)),
            pl.BlockSpec((1, 3, S), lambda i, j: (i, 0, 0)),
            pl.BlockSpec((1, 4, S), lambda i, j: (i, 0, 0)),
            pl.BlockSpec((1, S, 4), lambda i, j: (i, 0, 0)),
            pl.BlockSpec((1, S, D2), lambda i, j: (i, 0, 0)),
            pl.BlockSpec((1, TILE, D1), lambda i, j: (i, j, 0)),
            pl.BlockSpec((CIN, O0), lambda i, j: (0, 0)),
            pl.BlockSpec((1, O0), lambda i, j: (0, 0)),
        ],
        out_specs=[
            pl.BlockSpec((1, TILE, O0), lambda i, j: (i, j, 0)),
            pl.BlockSpec((1, 1, 8, O0), lambda i, j: (i, j, 0, 0)),
        ],
        out_shape=[
            jax.ShapeDtypeStruct((B, N, O0), f32),
            jax.ShapeDtypeStruct((B, NJ, 8, O0), f32),
        ],
    )(x1t, xyz2, ft, feature, p2t, p1t, w0t, b0r)
    y1, p1s = jax.lax.optimization_barrier((y1, p1s))

    import functools
    cnt = float(B * N)
    R8 = B * NJ * 8

    def stats(parts, g, be, O):
        return pl.pallas_call(
            functools.partial(_ks_body, cnt),
            in_specs=[
                pl.BlockSpec((R8, O), lambda: (0, 0)),
                pl.BlockSpec((1, O), lambda: (0, 0)),
                pl.BlockSpec((1, O), lambda: (0, 0)),
            ],
            out_specs=[
                pl.BlockSpec((1, O), lambda: (0, 0)),
                pl.BlockSpec((1, O), lambda: (0, 0)),
            ],
            out_shape=[
                jax.ShapeDtypeStruct((1, O), f32),
                jax.ShapeDtypeStruct((1, O), f32),
            ],
        )(parts.reshape(R8, O), g.reshape(1, O), be.reshape(1, O))

    a1, c1 = jax.lax.optimization_barrier(stats(p1s, g0, beta0, O0))

    y2, p2s = pl.pallas_call(
        _k2_body,
        grid=grid,
        in_specs=[
            pl.BlockSpec((1, TILE, O0), lambda i, j: (i, j, 0)),
            pl.BlockSpec((1, O0), lambda i, j: (0, 0)),
            pl.BlockSpec((1, O0), lambda i, j: (0, 0)),
            pl.BlockSpec((O0, O1), lambda i, j: (0, 0)),
            pl.BlockSpec((1, O1), lambda i, j: (0, 0)),
        ],
        out_specs=[
            pl.BlockSpec((1, TILE, O1), lambda i, j: (i, j, 0)),
            pl.BlockSpec((1, 1, 8, O1), lambda i, j: (i, j, 0, 0)),
        ],
        out_shape=[
            jax.ShapeDtypeStruct((B, N, O1), f32),
            jax.ShapeDtypeStruct((B, NJ, 8, O1), f32),
        ],
    )(y1, a1, c1, w1t, b1r)
    y2, p2s = jax.lax.optimization_barrier((y2, p2s))

    a2, c2 = jax.lax.optimization_barrier(stats(p2s, g1, beta1, O1))

    out = pl.pallas_call(
        _k3_body,
        grid=grid,
        in_specs=[
            pl.BlockSpec((1, TILE, O1), lambda i, j: (i, j, 0)),
            pl.BlockSpec((1, O1), lambda i, j: (0, 0)),
            pl.BlockSpec((1, O1), lambda i, j: (0, 0)),
        ],
        out_specs=pl.BlockSpec((1, O1, TILE), lambda i, j: (i, 0, j)),
        out_shape=jax.ShapeDtypeStruct((B, O1, N), f32),
    )(y2, a2, c2)

    return out
